# baseline (speedup 1.0000x reference)
import functools

import jax
import jax.numpy as jnp
from jax.experimental import pallas as pl
from jax.experimental.pallas import tpu as pltpu

HEADS = 4
K_DIM = 128
KNN = 8
N_KEYS = 25600
D_MODEL = 1024
BN_EPS = 1e-5


def _shared_swiglu_kernel(x_ref, w1_ref, w3_ref, w2_ref, o_ref):
    xb = x_ref[...]
    h1 = jnp.dot(xb, w1_ref[...].T, preferred_element_type=jnp.float32)
    h3 = jnp.dot(xb, w3_ref[...].T, preferred_element_type=jnp.float32)
    h = (h1 * jax.nn.sigmoid(h1)) * h3
    o_ref[...] = jnp.dot(h, w2_ref[...].T, preferred_element_type=jnp.float32)


def _shared_swiglu(x2d, s_w1, s_w2, s_w3, blk=256):
    bs, d = x2d.shape
    hid = s_w1.shape[0]
    grid = (bs // blk,)
    return pl.pallas_call(
        _shared_swiglu_kernel,
        grid=grid,
        in_specs=[
            pl.BlockSpec((blk, d), lambda i: (i, 0)),
            pl.BlockSpec((hid, d), lambda i: (0, 0)),
            pl.BlockSpec((hid, d), lambda i: (0, 0)),
            pl.BlockSpec((d, hid), lambda i: (0, 0)),
        ],
        out_specs=pl.BlockSpec((blk, d), lambda i: (i, 0)),
        out_shape=jax.ShapeDtypeStruct((bs, d), jnp.float32),
    )(x2d, s_w1, s_w3, s_w2)


def kernel(x, Wq, bq, bn_w, bn_b, bn_mean, bn_var, keys, w_down_table, w_up_table,
           a_w1, a_w2, a_w3, s_w1, s_w2, s_w3):
    b, t, d = x.shape
    bs = b * t
    x2d = x.reshape(bs, d)
    q = x2d @ Wq.T + bq
    q = (q - bn_mean) / jnp.sqrt(bn_var + BN_EPS) * bn_w + bn_b
    q = q.reshape(bs, HEADS, K_DIM)
    s_list, i_list = [], []
    for h in range(HEADS):
        sc = q[:, h, :] @ keys[h].T
        vals, idx = jax.lax.top_k(sc, KNN)
        s_list.append(vals)
        i_list.append(idx)
    scores = jnp.stack(s_list, axis=1)
    indices = jnp.stack(i_list, axis=1)
    scores = jax.nn.softmax(scores.reshape(-1, KNN).astype(jnp.float32), axis=-1)
    indices = indices.reshape(b, t, HEADS, KNN)
    scores = scores.reshape(b, t, HEADS, KNN)
    w_down = w_down_table[indices]
    w_up = w_up_table[indices]
    xm = jnp.einsum('btd,bthkd->bthk', x, w_down)
    xa = (jax.nn.silu(xm @ a_w1.T) * (xm @ a_w3.T)) @ a_w2.T
    x2 = xa * scores
    out = jnp.einsum('bthk,bthkd->btd', x2, w_up)
    shared = _shared_swiglu(x2d, s_w1, s_w2, s_w3).reshape(b, t, d)
    return out + shared


# speedup vs baseline: 2.0337x; 2.0337x over previous
import functools

import jax
import jax.numpy as jnp
from jax import lax
from jax.experimental import pallas as pl
from jax.experimental.pallas import tpu as pltpu

HEADS = 4
K_DIM = 128
KNN = 8
N_KEYS = 25600
D_MODEL = 1024
BN_EPS = 1e-5

T_BLK = 128
NEG = -3.0e38


def _score_topk_kernel(x_ref, wq_ref, bq_ref, keys_ref, vals_ref, idx_ref):
    xb = x_ref[...]
    wq = wq_ref[0]
    q = jnp.dot(xb, wq.T, preferred_element_type=jnp.float32) + bq_ref[0]
    keys_h = keys_ref[0]
    s = jnp.dot(q, keys_h.T, preferred_element_type=jnp.float32)
    iota = lax.broadcasted_iota(jnp.int32, (T_BLK, N_KEYS), 1)
    vals = []
    idxs = []
    for _ in range(KNN):
        m = jnp.max(s, axis=1)
        eq = s == m[:, None]
        am = jnp.min(jnp.where(eq, iota, N_KEYS), axis=1)
        s = jnp.where(iota == am[:, None], NEG, s)
        vals.append(m)
        idxs.append(am)
    v = jnp.stack(vals, axis=1)
    v = v - v[:, 0][:, None]
    e = jnp.exp(v)
    vals_ref[0] = e / jnp.sum(e, axis=1)[:, None]
    idx_ref[0] = jnp.stack(idxs, axis=1)


def _score_topk(x2d, wq_eff, b_eff, keys):
    bs, d = x2d.shape
    grid = (HEADS, bs // T_BLK)
    vals, idx = pl.pallas_call(
        _score_topk_kernel,
        grid=grid,
        in_specs=[
            pl.BlockSpec((T_BLK, d), lambda h, i: (i, 0)),
            pl.BlockSpec((1, K_DIM, d), lambda h, i: (h, 0, 0)),
            pl.BlockSpec((1, 1, K_DIM), lambda h, i: (h, 0, 0)),
            pl.BlockSpec((1, N_KEYS, K_DIM), lambda h, i: (h, 0, 0)),
        ],
        out_specs=[
            pl.BlockSpec((1, T_BLK, KNN), lambda h, i: (h, i, 0)),
            pl.BlockSpec((1, T_BLK, KNN), lambda h, i: (h, i, 0)),
        ],
        out_shape=[
            jax.ShapeDtypeStruct((HEADS, bs, KNN), jnp.float32),
            jax.ShapeDtypeStruct((HEADS, bs, KNN), jnp.int32),
        ],
    )(x2d, wq_eff, b_eff, keys)
    return vals, idx


def _shared_swiglu_kernel(x_ref, w1_ref, w3_ref, w2_ref, o_ref):
    xb = x_ref[...]
    h1 = jnp.dot(xb, w1_ref[...].T, preferred_element_type=jnp.float32)
    h3 = jnp.dot(xb, w3_ref[...].T, preferred_element_type=jnp.float32)
    h = (h1 * jax.nn.sigmoid(h1)) * h3
    o_ref[...] = jnp.dot(h, w2_ref[...].T, preferred_element_type=jnp.float32)


def _shared_swiglu(x2d, s_w1, s_w2, s_w3, blk=256):
    bs, d = x2d.shape
    hid = s_w1.shape[0]
    grid = (bs // blk,)
    return pl.pallas_call(
        _shared_swiglu_kernel,
        grid=grid,
        in_specs=[
            pl.BlockSpec((blk, d), lambda i: (i, 0)),
            pl.BlockSpec((hid, d), lambda i: (0, 0)),
            pl.BlockSpec((hid, d), lambda i: (0, 0)),
            pl.BlockSpec((d, hid), lambda i: (0, 0)),
        ],
        out_specs=pl.BlockSpec((blk, d), lambda i: (i, 0)),
        out_shape=jax.ShapeDtypeStruct((bs, d), jnp.float32),
    )(x2d, s_w1, s_w3, s_w2)


def kernel(x, Wq, bq, bn_w, bn_b, bn_mean, bn_var, keys, w_down_table, w_up_table,
           a_w1, a_w2, a_w3, s_w1, s_w2, s_w3):
    b, t, d = x.shape
    bs = b * t
    x2d = x.reshape(bs, d)

    scale = bn_w / jnp.sqrt(bn_var + BN_EPS)
    wq_eff = (Wq * scale[:, None]).reshape(HEADS, K_DIM, d)
    b_eff = (bq * scale + bn_b - bn_mean * scale).reshape(HEADS, 1, K_DIM)

    scores_h, idx_h = _score_topk(x2d, wq_eff, b_eff, keys)
    scores = scores_h.transpose(1, 0, 2).reshape(b, t, HEADS, KNN)
    indices = idx_h.transpose(1, 0, 2).reshape(b, t, HEADS, KNN)

    w_down = w_down_table[indices]
    w_up = w_up_table[indices]
    xm = jnp.einsum('btd,bthkd->bthk', x, w_down)
    xa = (jax.nn.silu(xm @ a_w1.T) * (xm @ a_w3.T)) @ a_w2.T
    x2 = xa * scores
    out = jnp.einsum('bthk,bthkd->btd', x2, w_up)
    shared = _shared_swiglu(x2d, s_w1, s_w2, s_w3).reshape(b, t, d)
    return out + shared


# speedup vs baseline: 4.3795x; 2.1535x over previous
import functools

import jax
import jax.numpy as jnp
from jax import lax
from jax.experimental import pallas as pl
from jax.experimental.pallas import tpu as pltpu
from jax.experimental.pallas import tpu_sc as plsc

HEADS = 4
K_DIM = 128
KNN = 8
N_KEYS = 25600
D_MODEL = 1024
BN_EPS = 1e-5

T_BLK = 128
NEG = -3.0e38

NW = 32
BS = 2048
TPW = BS // NW
KH = HEADS * KNN


def _score_topk_kernel(x_ref, wq_ref, bq_ref, keys_ref, vals_ref, idx_ref):
    xb = x_ref[...]
    wq = wq_ref[0]
    q = jnp.dot(xb, wq.T, preferred_element_type=jnp.float32) + bq_ref[0]
    keys_h = keys_ref[0]
    s = jnp.dot(q, keys_h.T, preferred_element_type=jnp.float32)
    iota = lax.broadcasted_iota(jnp.int32, (T_BLK, N_KEYS), 1)
    vals = []
    idxs = []
    for _ in range(KNN):
        m = jnp.max(s, axis=1)
        eq = s == m[:, None]
        am = jnp.min(jnp.where(eq, iota, N_KEYS), axis=1)
        s = jnp.where(iota == am[:, None], NEG, s)
        vals.append(m)
        idxs.append(am)
    v = jnp.stack(vals, axis=1)
    v = v - v[:, 0][:, None]
    e = jnp.exp(v)
    vals_ref[0] = e / jnp.sum(e, axis=1)[:, None]
    idx_ref[0] = jnp.stack(idxs, axis=1)


def _score_topk(x2d, wq_eff, b_eff, keys):
    bs, d = x2d.shape
    grid = (HEADS, bs // T_BLK)
    return pl.pallas_call(
        _score_topk_kernel,
        grid=grid,
        in_specs=[
            pl.BlockSpec((T_BLK, d), lambda h, i: (i, 0)),
            pl.BlockSpec((1, K_DIM, d), lambda h, i: (h, 0, 0)),
            pl.BlockSpec((1, 1, K_DIM), lambda h, i: (h, 0, 0)),
            pl.BlockSpec((1, N_KEYS, K_DIM), lambda h, i: (h, 0, 0)),
        ],
        out_specs=[
            pl.BlockSpec((1, T_BLK, KNN), lambda h, i: (h, i, 0)),
            pl.BlockSpec((1, T_BLK, KNN), lambda h, i: (h, i, 0)),
        ],
        out_shape=[
            jax.ShapeDtypeStruct((HEADS, bs, KNN), jnp.float32),
            jax.ShapeDtypeStruct((HEADS, bs, KNN), jnp.int32),
        ],
    )(x2d, wq_eff, b_eff, keys)


XM_KC = 8
XM_TB = 8


def _xm_all_kernel(x_ref, wd_ref, o_ref):
    o_ref[...] = jnp.dot(x_ref[...], wd_ref[...].T,
                         preferred_element_type=jnp.float32)


def _xm_all(x2d, w_down_table):
    bs, d = x2d.shape
    nk = w_down_table.shape[0]
    kb = nk // XM_KC
    tb = bs // XM_TB
    return pl.pallas_call(
        _xm_all_kernel,
        grid=(XM_KC, XM_TB),
        in_specs=[
            pl.BlockSpec((tb, d), lambda k, i: (i, 0)),
            pl.BlockSpec((kb, d), lambda k, i: (k, 0)),
        ],
        out_specs=pl.BlockSpec((tb, kb), lambda k, i: (i, k)),
        out_shape=jax.ShapeDtypeStruct((bs, nk), jnp.float32),
    )(x2d, w_down_table)


def _sc_gather_xm(xm_flat, flat_idx):
    mesh = plsc.VectorSubcoreMesh(core_axis_name="c", subcore_axis_name="s")
    npw = (BS * KH) // NW
    nch = npw // 128

    @functools.partial(
        pl.kernel,
        mesh=mesh,
        out_type=jax.ShapeDtypeStruct((BS * KH,), jnp.float32),
        scratch_types=[
            pltpu.VMEM((npw,), jnp.int32),
            pltpu.VMEM((npw,), jnp.float32),
            pltpu.SemaphoreType.DMA,
        ],
    )
    def k(table_hbm, idx_hbm, out_hbm, idx_v, val_v, sem):
        wid = lax.axis_index("s") * 2 + lax.axis_index("c")
        base = wid * npw
        pltpu.sync_copy(idx_hbm.at[pl.ds(base, npw)], idx_v)
        for i in range(nch):
            pltpu.async_copy(table_hbm.at[idx_v.at[pl.ds(i * 128, 128)]],
                             val_v.at[pl.ds(i * 128, 128)], sem)
        for i in range(nch):
            pltpu.make_async_copy(table_hbm.at[idx_v.at[pl.ds(i * 128, 128)]],
                                  val_v.at[pl.ds(i * 128, 128)], sem).wait()
        pltpu.sync_copy(val_v, out_hbm.at[pl.ds(base, npw)])

    return k(xm_flat, flat_idx)


def _sc_combine(w_up_table, idx2d, x2):
    mesh = plsc.VectorSubcoreMesh(core_axis_name="c", subcore_axis_name="s")
    d = w_up_table.shape[1]
    nchunks = d // 16

    @functools.partial(
        pl.kernel,
        mesh=mesh,
        out_type=jax.ShapeDtypeStruct((BS, d), jnp.float32),
        scratch_types=[
            pltpu.VMEM((TPW, KH), jnp.int32),
            pltpu.VMEM((TPW, KH), jnp.float32),
            pltpu.VMEM((KH, d), jnp.float32),
            pltpu.VMEM((KH, d), jnp.float32),
            pltpu.VMEM((d,), jnp.float32),
            pltpu.SemaphoreType.DMA,
            pltpu.SemaphoreType.DMA,
        ],
    )
    def k(wup_hbm, idx_hbm, x2_hbm, out_hbm, idx_v, x2_v, rows_a, rows_b,
          acc_v, sem_a, sem_b):
        wid = lax.axis_index("s") * 2 + lax.axis_index("c")
        base = wid * TPW
        pltpu.sync_copy(idx_hbm.at[pl.ds(base, TPW)], idx_v)
        pltpu.sync_copy(x2_hbm.at[pl.ds(base, TPW)], x2_v)

        pltpu.async_copy(wup_hbm.at[idx_v.at[0]], rows_a, sem_a)

        def body(t, carry):
            del carry
            even = lax.rem(t, 2) == 0

            def compute(rows_v, sem, other_rows, other_sem):
                pltpu.make_async_copy(wup_hbm.at[idx_v.at[t]], rows_v,
                                      sem).wait()

                @pl.when(t + 1 < TPW)
                def _():
                    pltpu.async_copy(wup_hbm.at[idx_v.at[t + 1]], other_rows,
                                     other_sem)

                xrow0 = x2_v[t, pl.ds(0, 16)]
                xrow1 = x2_v[t, pl.ds(16, 16)]
                xvs = ([jnp.full((16,), xrow0[j], jnp.float32)
                        for j in range(16)]
                       + [jnp.full((16,), xrow1[j], jnp.float32)
                          for j in range(16)])

                def chunk(c, carry2):
                    del carry2
                    off = pl.multiple_of(c * 16, 16)
                    acc = jnp.zeros((16,), jnp.float32)
                    for j in range(KH):
                        acc = acc + xvs[j] * rows_v[j, pl.ds(off, 16)]
                    acc_v[pl.ds(off, 16)] = acc
                    return 0

                lax.fori_loop(0, nchunks, chunk, 0, unroll=2)
                pltpu.sync_copy(acc_v, out_hbm.at[base + t])

            @pl.when(even)
            def _():
                compute(rows_a, sem_a, rows_b, sem_b)

            @pl.when(jnp.logical_not(even))
            def _():
                compute(rows_b, sem_b, rows_a, sem_a)

            return 0

        lax.fori_loop(0, TPW, body, 0)

    return k(w_up_table, idx2d, x2)


def _shared_swiglu_kernel(x_ref, w1_ref, w3_ref, w2_ref, o_ref):
    xb = x_ref[...]
    h1 = jnp.dot(xb, w1_ref[...].T, preferred_element_type=jnp.float32)
    h3 = jnp.dot(xb, w3_ref[...].T, preferred_element_type=jnp.float32)
    h = (h1 * jax.nn.sigmoid(h1)) * h3
    o_ref[...] = jnp.dot(h, w2_ref[...].T, preferred_element_type=jnp.float32)


def _shared_swiglu(x2d, s_w1, s_w2, s_w3, blk=256):
    bs, d = x2d.shape
    hid = s_w1.shape[0]
    grid = (bs // blk,)
    return pl.pallas_call(
        _shared_swiglu_kernel,
        grid=grid,
        in_specs=[
            pl.BlockSpec((blk, d), lambda i: (i, 0)),
            pl.BlockSpec((hid, d), lambda i: (0, 0)),
            pl.BlockSpec((hid, d), lambda i: (0, 0)),
            pl.BlockSpec((d, hid), lambda i: (0, 0)),
        ],
        out_specs=pl.BlockSpec((blk, d), lambda i: (i, 0)),
        out_shape=jax.ShapeDtypeStruct((bs, d), jnp.float32),
    )(x2d, s_w1, s_w3, s_w2)


def kernel(x, Wq, bq, bn_w, bn_b, bn_mean, bn_var, keys, w_down_table, w_up_table,
           a_w1, a_w2, a_w3, s_w1, s_w2, s_w3):
    b, t, d = x.shape
    bs = b * t
    x2d = x.reshape(bs, d)

    scale = bn_w / jnp.sqrt(bn_var + BN_EPS)
    wq_eff = (Wq * scale[:, None]).reshape(HEADS, K_DIM, d)
    b_eff = (bq * scale + bn_b - bn_mean * scale).reshape(HEADS, 1, K_DIM)

    scores_h, idx_h = _score_topk(x2d, wq_eff, b_eff, keys)
    scores = scores_h.transpose(1, 0, 2).reshape(bs, KH)
    idx2d = idx_h.transpose(1, 0, 2).reshape(bs, KH)

    xm_all = _xm_all(x2d, w_down_table)
    flat_idx = (jnp.arange(bs, dtype=jnp.int32)[:, None] * N_KEYS
                + idx2d).reshape(-1)
    xm = _sc_gather_xm(xm_all.reshape(-1), flat_idx).reshape(bs, HEADS, KNN)

    xa = (jax.nn.silu(xm @ a_w1.T) * (xm @ a_w3.T)) @ a_w2.T
    x2 = (xa.reshape(bs, KH) * scores).astype(jnp.float32)

    out = _sc_combine(w_up_table, idx2d, x2)
    shared = _shared_swiglu(x2d, s_w1, s_w2, s_w3)
    return (out + shared).reshape(b, t, d)


# speedup vs baseline: 6.6900x; 1.5276x over previous
import functools

import jax
import jax.numpy as jnp
from jax import lax
from jax.experimental import pallas as pl
from jax.experimental.pallas import tpu as pltpu
from jax.experimental.pallas import tpu_sc as plsc

HEADS = 4
K_DIM = 128
KNN = 8
N_KEYS = 25600
D_MODEL = 1024
BN_EPS = 1e-5

T_BLK = 128
NEG = -3.0e38

NW = 32
BS = 2048
TPW = BS // NW
KH = HEADS * KNN


NCH = 1600
CH_W = N_KEYS // NCH


def _score_chunk_kernel(x_ref, wq_ref, bq_ref, keys_ref, s_ref, cm_ref):
    xb = x_ref[...]
    wq = wq_ref[0]
    q = jnp.dot(xb, wq.T, preferred_element_type=jnp.float32) + bq_ref[0]
    keys_h = keys_ref[0]
    s = jnp.dot(q, keys_h.T, preferred_element_type=jnp.float32)
    s_ref[0] = s
    p = s[:, 0:NCH]
    for j in range(1, CH_W):
        p = jnp.maximum(p, s[:, j * NCH:(j + 1) * NCH])
    iota = lax.broadcasted_iota(jnp.int32, (T_BLK, NCH), 1)
    cms = []
    for _ in range(KNN):
        m = jnp.max(p, axis=1)
        eq = p == m[:, None]
        cm = jnp.min(jnp.where(eq, iota, NCH), axis=1)
        p = jnp.where(iota == cm[:, None], NEG, p)
        cms.append(cm)
    cm_ref[0] = jnp.stack(cms, axis=1)


def _score_chunks(x2d, wq_eff, b_eff, keys):
    bs, d = x2d.shape
    grid = (HEADS, bs // T_BLK)
    return pl.pallas_call(
        _score_chunk_kernel,
        grid=grid,
        in_specs=[
            pl.BlockSpec((T_BLK, d), lambda h, i: (i, 0)),
            pl.BlockSpec((1, K_DIM, d), lambda h, i: (h, 0, 0)),
            pl.BlockSpec((1, 1, K_DIM), lambda h, i: (h, 0, 0)),
            pl.BlockSpec((1, N_KEYS, K_DIM), lambda h, i: (h, 0, 0)),
        ],
        out_specs=[
            pl.BlockSpec((1, T_BLK, N_KEYS), lambda h, i: (h, i, 0)),
            pl.BlockSpec((1, T_BLK, KNN), lambda h, i: (h, i, 0)),
        ],
        out_shape=[
            jax.ShapeDtypeStruct((HEADS, bs, N_KEYS), jnp.float32),
            jax.ShapeDtypeStruct((HEADS, bs, KNN), jnp.int32),
        ],
    )(x2d, wq_eff, b_eff, keys)


NTH = HEADS * BS
IPW = NTH // NW
NCAND = KNN * CH_W


def _sc_select(s_flat, chunks16):
    mesh = plsc.VectorSubcoreMesh(core_axis_name="c", subcore_axis_name="s")

    gdn = lax.GatherDimensionNumbers(offset_dims=(), collapsed_slice_dims=(0,),
                                     start_index_map=(0,))

    def _shuf(v, perm):
        return lax.gather(v, perm[:, None], gdn, slice_sizes=(1,),
                          mode=lax.GatherScatterMode.PROMISE_IN_BOUNDS)

    def _tree(v, op, perms):
        for p in perms:
            v = op(v, _shuf(v, p))
        return v

    @functools.partial(
        pl.kernel,
        mesh=mesh,
        out_type=[
            jax.ShapeDtypeStruct((NTH, 16), jnp.float32),
            jax.ShapeDtypeStruct((NTH, 16), jnp.int32),
        ],
        scratch_types=[
            pltpu.VMEM((IPW, 16), jnp.int32),
            pltpu.VMEM((NCAND,), jnp.int32),
            pltpu.VMEM((NCAND,), jnp.int32),
            pltpu.VMEM((NCAND,), jnp.float32),
            pltpu.VMEM((NCAND,), jnp.float32),
            pltpu.VMEM((IPW, 16), jnp.float32),
            pltpu.VMEM((IPW, 16), jnp.int32),
            pltpu.SemaphoreType.DMA,
            pltpu.SemaphoreType.DMA,
        ],
    )
    def k(s_hbm, ch_hbm, osc_hbm, oidx_hbm, ch_v, ia_v, ib_v, va_v, vb_v,
          osc_v, oidx_v, sem_a, sem_b):
        wid = lax.axis_index("s") * 2 + lax.axis_index("c")
        base = wid * IPW
        pltpu.sync_copy(ch_hbm.at[pl.ds(base, IPW)], ch_v)
        lane = lax.iota(jnp.int32, 16)
        stride = lane * NCH
        perms = [lane ^ (1 << b) for b in range(4)]

        def build(t, idx_v):
            crow = ch_v[t, pl.ds(0, 16)]
            sbase = (base + t) * N_KEYS
            for j in range(KNN):
                idx_v[pl.ds(j * 16, 16)] = stride + (sbase + crow[j])

        def fire(t, idx_v, val_v, sem):
            build(t, idx_v)
            pltpu.async_copy(s_hbm.at[idx_v], val_v, sem)

        fire(0, ia_v, va_v, sem_a)

        def body(t, carry):
            del carry
            even = lax.rem(t, 2) == 0

            def work(idx_v, val_v, sem, oidx2_v, oval2_v, osem):
                pltpu.make_async_copy(s_hbm.at[idx_v], val_v, sem).wait()

                @pl.when(t + 1 < IPW)
                def _():
                    fire(t + 1, oidx2_v, oval2_v, osem)

                crow = ch_v[t, pl.ds(0, 16)]
                vs = [val_v[pl.ds(j * 16, 16)] for j in range(KNN)]
                ks = [stride + crow[j] for j in range(KNN)]
                wv = jnp.full((16,), NEG, jnp.float32)
                wi = jnp.zeros((16,), jnp.int32)
                big = jnp.full((16,), jnp.int32(2 ** 30), jnp.int32)
                negs = jnp.full((16,), NEG, jnp.float32)
                for it in range(KNN):
                    m01 = jnp.maximum(vs[0], vs[1])
                    m23 = jnp.maximum(vs[2], vs[3])
                    m45 = jnp.maximum(vs[4], vs[5])
                    m67 = jnp.maximum(vs[6], vs[7])
                    m = jnp.maximum(jnp.maximum(m01, m23),
                                    jnp.maximum(m45, m67))
                    msp = _tree(m, jnp.maximum, perms)
                    sels = [jnp.where(vs[j] == msp, ks[j], big)
                            for j in range(KNN)]
                    s01 = jnp.minimum(sels[0], sels[1])
                    s23 = jnp.minimum(sels[2], sels[3])
                    s45 = jnp.minimum(sels[4], sels[5])
                    s67 = jnp.minimum(sels[6], sels[7])
                    sm = jnp.minimum(jnp.minimum(s01, s23),
                                     jnp.minimum(s45, s67))
                    wsp = _tree(sm, jnp.minimum, perms)
                    for j in range(KNN):
                        vs[j] = jnp.where(ks[j] == wsp, negs, vs[j])
                    hit = lane == it
                    wv = jnp.where(hit, msp, wv)
                    wi = jnp.where(hit, wsp, wi)
                e = jnp.exp(wv - _tree(wv, jnp.maximum, perms))
                ssum = _tree(e, jnp.add, perms)
                osc_v[t, pl.ds(0, 16)] = e / ssum
                oidx_v[t, pl.ds(0, 16)] = wi

            @pl.when(even)
            def _():
                work(ia_v, va_v, sem_a, ib_v, vb_v, sem_b)

            @pl.when(jnp.logical_not(even))
            def _():
                work(ib_v, vb_v, sem_b, ia_v, va_v, sem_a)

            return 0

        lax.fori_loop(0, IPW, body, 0)
        pltpu.sync_copy(osc_v, osc_hbm.at[pl.ds(base, IPW)])
        pltpu.sync_copy(oidx_v, oidx_hbm.at[pl.ds(base, IPW)])

    return k(s_flat, chunks16)


XM_KC = 8
XM_TB = 8


def _xm_all_kernel(x_ref, wd_ref, o_ref):
    o_ref[...] = jnp.dot(x_ref[...], wd_ref[...].T,
                         preferred_element_type=jnp.float32)


def _xm_all(x2d, w_down_table):
    bs, d = x2d.shape
    nk = w_down_table.shape[0]
    kb = nk // XM_KC
    tb = bs // XM_TB
    return pl.pallas_call(
        _xm_all_kernel,
        grid=(XM_KC, XM_TB),
        in_specs=[
            pl.BlockSpec((tb, d), lambda k, i: (i, 0)),
            pl.BlockSpec((kb, d), lambda k, i: (k, 0)),
        ],
        out_specs=pl.BlockSpec((tb, kb), lambda k, i: (i, k)),
        out_shape=jax.ShapeDtypeStruct((bs, nk), jnp.float32),
    )(x2d, w_down_table)


def _sc_gather_xm(xm_flat, flat_idx):
    mesh = plsc.VectorSubcoreMesh(core_axis_name="c", subcore_axis_name="s")
    npw = (BS * KH) // NW
    nch = npw // 128

    @functools.partial(
        pl.kernel,
        mesh=mesh,
        out_type=jax.ShapeDtypeStruct((BS * KH,), jnp.float32),
        scratch_types=[
            pltpu.VMEM((npw,), jnp.int32),
            pltpu.VMEM((npw,), jnp.float32),
            pltpu.SemaphoreType.DMA,
        ],
    )
    def k(table_hbm, idx_hbm, out_hbm, idx_v, val_v, sem):
        wid = lax.axis_index("s") * 2 + lax.axis_index("c")
        base = wid * npw
        pltpu.sync_copy(idx_hbm.at[pl.ds(base, npw)], idx_v)
        for i in range(nch):
            pltpu.async_copy(table_hbm.at[idx_v.at[pl.ds(i * 128, 128)]],
                             val_v.at[pl.ds(i * 128, 128)], sem)
        for i in range(nch):
            pltpu.make_async_copy(table_hbm.at[idx_v.at[pl.ds(i * 128, 128)]],
                                  val_v.at[pl.ds(i * 128, 128)], sem).wait()
        pltpu.sync_copy(val_v, out_hbm.at[pl.ds(base, npw)])

    return k(xm_flat, flat_idx)


def _sc_combine(w_up_table, idx2d, x2):
    mesh = plsc.VectorSubcoreMesh(core_axis_name="c", subcore_axis_name="s")
    d = w_up_table.shape[1]
    nchunks = d // 16

    @functools.partial(
        pl.kernel,
        mesh=mesh,
        out_type=jax.ShapeDtypeStruct((BS, d), jnp.float32),
        scratch_types=[
            pltpu.VMEM((TPW, KH), jnp.int32),
            pltpu.VMEM((TPW, KH), jnp.float32),
            pltpu.VMEM((KH, d), jnp.float32),
            pltpu.VMEM((KH, d), jnp.float32),
            pltpu.VMEM((d,), jnp.float32),
            pltpu.SemaphoreType.DMA,
            pltpu.SemaphoreType.DMA,
        ],
    )
    def k(wup_hbm, idx_hbm, x2_hbm, out_hbm, idx_v, x2_v, rows_a, rows_b,
          acc_v, sem_a, sem_b):
        wid = lax.axis_index("s") * 2 + lax.axis_index("c")
        base = wid * TPW
        pltpu.sync_copy(idx_hbm.at[pl.ds(base, TPW)], idx_v)
        pltpu.sync_copy(x2_hbm.at[pl.ds(base, TPW)], x2_v)

        pltpu.async_copy(wup_hbm.at[idx_v.at[0]], rows_a, sem_a)

        def body(t, carry):
            del carry
            even = lax.rem(t, 2) == 0

            def compute(rows_v, sem, other_rows, other_sem):
                pltpu.make_async_copy(wup_hbm.at[idx_v.at[t]], rows_v,
                                      sem).wait()

                @pl.when(t + 1 < TPW)
                def _():
                    pltpu.async_copy(wup_hbm.at[idx_v.at[t + 1]], other_rows,
                                     other_sem)

                xrow0 = x2_v[t, pl.ds(0, 16)]
                xrow1 = x2_v[t, pl.ds(16, 16)]
                xvs = ([jnp.full((16,), xrow0[j], jnp.float32)
                        for j in range(16)]
                       + [jnp.full((16,), xrow1[j], jnp.float32)
                          for j in range(16)])

                def chunk(c, carry2):
                    del carry2
                    off = pl.multiple_of(c * 16, 16)
                    acc = jnp.zeros((16,), jnp.float32)
                    for j in range(KH):
                        acc = acc + xvs[j] * rows_v[j, pl.ds(off, 16)]
                    acc_v[pl.ds(off, 16)] = acc
                    return 0

                lax.fori_loop(0, nchunks, chunk, 0, unroll=2)
                pltpu.sync_copy(acc_v, out_hbm.at[base + t])

            @pl.when(even)
            def _():
                compute(rows_a, sem_a, rows_b, sem_b)

            @pl.when(jnp.logical_not(even))
            def _():
                compute(rows_b, sem_b, rows_a, sem_a)

            return 0

        lax.fori_loop(0, TPW, body, 0)

    return k(w_up_table, idx2d, x2)


def _shared_swiglu_kernel(x_ref, w1_ref, w3_ref, w2_ref, o_ref):
    xb = x_ref[...]
    h1 = jnp.dot(xb, w1_ref[...].T, preferred_element_type=jnp.float32)
    h3 = jnp.dot(xb, w3_ref[...].T, preferred_element_type=jnp.float32)
    h = (h1 * jax.nn.sigmoid(h1)) * h3
    o_ref[...] = jnp.dot(h, w2_ref[...].T, preferred_element_type=jnp.float32)


def _shared_swiglu(x2d, s_w1, s_w2, s_w3, blk=256):
    bs, d = x2d.shape
    hid = s_w1.shape[0]
    grid = (bs // blk,)
    return pl.pallas_call(
        _shared_swiglu_kernel,
        grid=grid,
        in_specs=[
            pl.BlockSpec((blk, d), lambda i: (i, 0)),
            pl.BlockSpec((hid, d), lambda i: (0, 0)),
            pl.BlockSpec((hid, d), lambda i: (0, 0)),
            pl.BlockSpec((d, hid), lambda i: (0, 0)),
        ],
        out_specs=pl.BlockSpec((blk, d), lambda i: (i, 0)),
        out_shape=jax.ShapeDtypeStruct((bs, d), jnp.float32),
    )(x2d, s_w1, s_w3, s_w2)


def kernel(x, Wq, bq, bn_w, bn_b, bn_mean, bn_var, keys, w_down_table, w_up_table,
           a_w1, a_w2, a_w3, s_w1, s_w2, s_w3):
    b, t, d = x.shape
    bs = b * t
    x2d = x.reshape(bs, d)

    scale = bn_w / jnp.sqrt(bn_var + BN_EPS)
    wq_eff = (Wq * scale[:, None]).reshape(HEADS, K_DIM, d)
    b_eff = (bq * scale + bn_b - bn_mean * scale).reshape(HEADS, 1, K_DIM)

    s_full, cm8 = _score_chunks(x2d, wq_eff, b_eff, keys)
    chunks16 = jnp.pad(cm8, ((0, 0), (0, 0), (0, 16 - KNN))).reshape(NTH, 16)
    sc_scores, sc_idx = _sc_select(s_full.reshape(-1), chunks16)
    scores_h = sc_scores.reshape(HEADS, bs, 16)[:, :, :KNN]
    idx_h = sc_idx.reshape(HEADS, bs, 16)[:, :, :KNN]
    scores = scores_h.transpose(1, 0, 2).reshape(bs, KH)
    idx2d = idx_h.transpose(1, 0, 2).reshape(bs, KH)

    xm_all = _xm_all(x2d, w_down_table)
    flat_idx = (jnp.arange(bs, dtype=jnp.int32)[:, None] * N_KEYS
                + idx2d).reshape(-1)
    xm = _sc_gather_xm(xm_all.reshape(-1), flat_idx).reshape(bs, HEADS, KNN)

    xa = (jax.nn.silu(xm @ a_w1.T) * (xm @ a_w3.T)) @ a_w2.T
    x2 = (xa.reshape(bs, KH) * scores).astype(jnp.float32)

    out = _sc_combine(w_up_table, idx2d, x2)
    shared = _shared_swiglu(x2d, s_w1, s_w2, s_w3)
    return (out + shared).reshape(b, t, d)


# speedup vs baseline: 9.4028x; 1.4055x over previous
import functools

import jax
import jax.numpy as jnp
from jax import lax
from jax.experimental import pallas as pl
from jax.experimental.pallas import tpu as pltpu
from jax.experimental.pallas import tpu_sc as plsc

HEADS = 4
K_DIM = 128
KNN = 8
N_KEYS = 25600
D_MODEL = 1024
BN_EPS = 1e-5

T_BLK = 128
NEG = -3.0e38

NW = 32
BS = 2048
TPW = BS // NW
KH = HEADS * KNN


NCH = 1600
CH_W = N_KEYS // NCH


def _score_chunk_kernel(x_ref, wq_ref, bq_ref, keys_ref, s_ref, cm_ref):
    xb = x_ref[...]
    wq = wq_ref[0]
    q = jnp.dot(xb, wq.T, preferred_element_type=jnp.float32) + bq_ref[0]
    keys_h = keys_ref[0]
    s = jnp.dot(q, keys_h.T, preferred_element_type=jnp.float32)
    s_ref[0] = s.reshape(T_BLK // 8, 8, N_KEYS // 128, 128).transpose(0, 2, 1, 3)
    p = s[:, 0:NCH]
    for j in range(1, CH_W):
        p = jnp.maximum(p, s[:, j * NCH:(j + 1) * NCH])
    iota = lax.broadcasted_iota(jnp.int32, (T_BLK, NCH), 1)
    cms = []
    for _ in range(KNN):
        m = jnp.max(p, axis=1)
        eq = p == m[:, None]
        cm = jnp.min(jnp.where(eq, iota, NCH), axis=1)
        p = jnp.where(iota == cm[:, None], NEG, p)
        cms.append(cm)
    cm_ref[0] = jnp.stack(cms, axis=1)


def _score_chunks(x2d, wq_eff, b_eff, keys):
    bs, d = x2d.shape
    grid = (HEADS, bs // T_BLK)
    return pl.pallas_call(
        _score_chunk_kernel,
        grid=grid,
        in_specs=[
            pl.BlockSpec((T_BLK, d), lambda h, i: (i, 0)),
            pl.BlockSpec((1, K_DIM, d), lambda h, i: (h, 0, 0)),
            pl.BlockSpec((1, 1, K_DIM), lambda h, i: (h, 0, 0)),
            pl.BlockSpec((1, N_KEYS, K_DIM), lambda h, i: (h, 0, 0)),
        ],
        out_specs=[
            pl.BlockSpec((1, T_BLK // 8, N_KEYS // 128, 8, 128),
                         lambda h, i: (h, i, 0, 0, 0)),
            pl.BlockSpec((1, T_BLK, KNN), lambda h, i: (h, i, 0)),
        ],
        out_shape=[
            jax.ShapeDtypeStruct((HEADS, bs // 8, N_KEYS // 128, 8, 128),
                                 jnp.float32),
            jax.ShapeDtypeStruct((HEADS, bs, KNN), jnp.int32),
        ],
    )(x2d, wq_eff, b_eff, keys)


NTH = HEADS * BS
IPW = NTH // NW
NCAND = KNN * CH_W


def _sc_select(s_flat, chunks16):
    mesh = plsc.VectorSubcoreMesh(core_axis_name="c", subcore_axis_name="s")

    gdn = lax.GatherDimensionNumbers(offset_dims=(), collapsed_slice_dims=(0,),
                                     start_index_map=(0,))

    def _shuf(v, perm):
        return lax.gather(v, perm[:, None], gdn, slice_sizes=(1,),
                          mode=lax.GatherScatterMode.PROMISE_IN_BOUNDS)

    def _tree(v, op, perms):
        for p in perms:
            v = op(v, _shuf(v, p))
        return v

    @functools.partial(
        pl.kernel,
        mesh=mesh,
        out_type=[
            jax.ShapeDtypeStruct((NTH, 16), jnp.float32),
            jax.ShapeDtypeStruct((NTH, 16), jnp.int32),
        ],
        scratch_types=[
            pltpu.VMEM((IPW, 16), jnp.int32),
            pltpu.VMEM((NCAND,), jnp.int32),
            pltpu.VMEM((NCAND,), jnp.int32),
            pltpu.VMEM((NCAND,), jnp.float32),
            pltpu.VMEM((NCAND,), jnp.float32),
            pltpu.VMEM((IPW, 16), jnp.float32),
            pltpu.VMEM((IPW, 16), jnp.int32),
            pltpu.SemaphoreType.DMA,
            pltpu.SemaphoreType.DMA,
        ],
    )
    def k(s_hbm, ch_hbm, osc_hbm, oidx_hbm, ch_v, ia_v, ib_v, va_v, vb_v,
          osc_v, oidx_v, sem_a, sem_b):
        wid = lax.axis_index("s") * 2 + lax.axis_index("c")
        base = wid * IPW
        pltpu.sync_copy(ch_hbm.at[pl.ds(base, IPW)], ch_v)
        lane = lax.iota(jnp.int32, 16)
        stride = lane * NCH
        perms = [lane ^ (1 << b) for b in range(4)]

        def build(t, idx_v):
            crow = ch_v[t, pl.ds(0, 16)]
            u = base + t
            h = lax.shift_right_logical(u, 11)
            tok = jnp.bitwise_and(u, 2047)
            a = ((h * 256 + lax.shift_right_logical(tok, 3)) * 204800
                 + jnp.bitwise_and(tok, 7) * 128)
            for j in range(KNN):
                kvec = stride + crow[j]
                kc = lax.shift_right_logical(kvec, 7)
                kl = jnp.bitwise_and(kvec, 127)
                idx_v[pl.ds(j * 16, 16)] = a + kc * 1024 + kl

        def fire(t, idx_v, val_v, sem):
            build(t, idx_v)
            pltpu.async_copy(s_hbm.at[idx_v], val_v, sem)

        fire(0, ia_v, va_v, sem_a)

        def body(t, carry):
            del carry
            even = lax.rem(t, 2) == 0

            def work(idx_v, val_v, sem, oidx2_v, oval2_v, osem):
                pltpu.make_async_copy(s_hbm.at[idx_v], val_v, sem).wait()

                @pl.when(t + 1 < IPW)
                def _():
                    fire(t + 1, oidx2_v, oval2_v, osem)

                crow = ch_v[t, pl.ds(0, 16)]
                vs = [val_v[pl.ds(j * 16, 16)] for j in range(KNN)]
                ks = [stride + crow[j] for j in range(KNN)]
                wv = jnp.full((16,), NEG, jnp.float32)
                wi = jnp.zeros((16,), jnp.int32)
                big = jnp.full((16,), jnp.int32(2 ** 30), jnp.int32)
                negs = jnp.full((16,), NEG, jnp.float32)
                for it in range(KNN):
                    m01 = jnp.maximum(vs[0], vs[1])
                    m23 = jnp.maximum(vs[2], vs[3])
                    m45 = jnp.maximum(vs[4], vs[5])
                    m67 = jnp.maximum(vs[6], vs[7])
                    m = jnp.maximum(jnp.maximum(m01, m23),
                                    jnp.maximum(m45, m67))
                    msp = _tree(m, jnp.maximum, perms)
                    sels = [jnp.where(vs[j] == msp, ks[j], big)
                            for j in range(KNN)]
                    s01 = jnp.minimum(sels[0], sels[1])
                    s23 = jnp.minimum(sels[2], sels[3])
                    s45 = jnp.minimum(sels[4], sels[5])
                    s67 = jnp.minimum(sels[6], sels[7])
                    sm = jnp.minimum(jnp.minimum(s01, s23),
                                     jnp.minimum(s45, s67))
                    wsp = _tree(sm, jnp.minimum, perms)
                    for j in range(KNN):
                        vs[j] = jnp.where(ks[j] == wsp, negs, vs[j])
                    hit = lane == it
                    wv = jnp.where(hit, msp, wv)
                    wi = jnp.where(hit, wsp, wi)
                e = jnp.exp(wv - _tree(wv, jnp.maximum, perms))
                ssum = _tree(e, jnp.add, perms)
                osc_v[t, pl.ds(0, 16)] = e / ssum
                oidx_v[t, pl.ds(0, 16)] = wi

            @pl.when(even)
            def _():
                work(ia_v, va_v, sem_a, ib_v, vb_v, sem_b)

            @pl.when(jnp.logical_not(even))
            def _():
                work(ib_v, vb_v, sem_b, ia_v, va_v, sem_a)

            return 0

        lax.fori_loop(0, IPW, body, 0)
        pltpu.sync_copy(osc_v, osc_hbm.at[pl.ds(base, IPW)])
        pltpu.sync_copy(oidx_v, oidx_hbm.at[pl.ds(base, IPW)])

    return k(s_flat, chunks16)


XM_KC = 8
XM_TB = 8


def _xm_all_kernel(x_ref, wd_ref, o_ref):
    tb, kb = o_ref.shape[0] * 8, o_ref.shape[1] * 128
    m = jnp.dot(x_ref[...], wd_ref[...].T, preferred_element_type=jnp.float32)
    o_ref[...] = m.reshape(tb // 8, 8, kb // 128, 128).transpose(0, 2, 1, 3)


def _xm_all(x2d, w_down_table):
    bs, d = x2d.shape
    nk = w_down_table.shape[0]
    kb = nk // XM_KC
    tb = bs // XM_TB
    return pl.pallas_call(
        _xm_all_kernel,
        grid=(XM_KC, XM_TB),
        in_specs=[
            pl.BlockSpec((tb, d), lambda k, i: (i, 0)),
            pl.BlockSpec((kb, d), lambda k, i: (k, 0)),
        ],
        out_specs=pl.BlockSpec((tb // 8, kb // 128, 8, 128),
                               lambda k, i: (i, k, 0, 0)),
        out_shape=jax.ShapeDtypeStruct((bs // 8, nk // 128, 8, 128),
                                       jnp.float32),
    )(x2d, w_down_table)


def _sc_gather_xm(xm_flat, flat_idx):
    mesh = plsc.VectorSubcoreMesh(core_axis_name="c", subcore_axis_name="s")
    npw = (BS * KH) // NW
    nch = npw // 128

    @functools.partial(
        pl.kernel,
        mesh=mesh,
        out_type=jax.ShapeDtypeStruct((BS * KH,), jnp.float32),
        scratch_types=[
            pltpu.VMEM((npw,), jnp.int32),
            pltpu.VMEM((npw,), jnp.float32),
            pltpu.SemaphoreType.DMA,
        ],
    )
    def k(table_hbm, idx_hbm, out_hbm, idx_v, val_v, sem):
        wid = lax.axis_index("s") * 2 + lax.axis_index("c")
        base = wid * npw
        pltpu.sync_copy(idx_hbm.at[pl.ds(base, npw)], idx_v)
        for i in range(nch):
            pltpu.async_copy(table_hbm.at[idx_v.at[pl.ds(i * 128, 128)]],
                             val_v.at[pl.ds(i * 128, 128)], sem)
        for i in range(nch):
            pltpu.make_async_copy(table_hbm.at[idx_v.at[pl.ds(i * 128, 128)]],
                                  val_v.at[pl.ds(i * 128, 128)], sem).wait()
        pltpu.sync_copy(val_v, out_hbm.at[pl.ds(base, npw)])

    return k(xm_flat, flat_idx)


def _sc_combine(w_up_table, idx2d, x2):
    mesh = plsc.VectorSubcoreMesh(core_axis_name="c", subcore_axis_name="s")
    d = w_up_table.shape[1]
    nchunks = d // 16

    @functools.partial(
        pl.kernel,
        mesh=mesh,
        out_type=jax.ShapeDtypeStruct((BS, d), jnp.float32),
        scratch_types=[
            pltpu.VMEM((TPW, KH), jnp.int32),
            pltpu.VMEM((TPW, KH), jnp.float32),
            pltpu.VMEM((KH, d), jnp.float32),
            pltpu.VMEM((KH, d), jnp.float32),
            pltpu.VMEM((d,), jnp.float32),
            pltpu.SemaphoreType.DMA,
            pltpu.SemaphoreType.DMA,
        ],
    )
    def k(wup_hbm, idx_hbm, x2_hbm, out_hbm, idx_v, x2_v, rows_a, rows_b,
          acc_v, sem_a, sem_b):
        wid = lax.axis_index("s") * 2 + lax.axis_index("c")
        base = wid * TPW
        pltpu.sync_copy(idx_hbm.at[pl.ds(base, TPW)], idx_v)
        pltpu.sync_copy(x2_hbm.at[pl.ds(base, TPW)], x2_v)

        pltpu.async_copy(wup_hbm.at[idx_v.at[0]], rows_a, sem_a)

        def body(t, carry):
            del carry
            even = lax.rem(t, 2) == 0

            def compute(rows_v, sem, other_rows, other_sem):
                pltpu.make_async_copy(wup_hbm.at[idx_v.at[t]], rows_v,
                                      sem).wait()

                @pl.when(t + 1 < TPW)
                def _():
                    pltpu.async_copy(wup_hbm.at[idx_v.at[t + 1]], other_rows,
                                     other_sem)

                xrow0 = x2_v[t, pl.ds(0, 16)]
                xrow1 = x2_v[t, pl.ds(16, 16)]
                xvs = ([jnp.full((16,), xrow0[j], jnp.float32)
                        for j in range(16)]
                       + [jnp.full((16,), xrow1[j], jnp.float32)
                          for j in range(16)])

                def chunk(c, carry2):
                    del carry2
                    off = pl.multiple_of(c * 16, 16)
                    acc = jnp.zeros((16,), jnp.float32)
                    for j in range(KH):
                        acc = acc + xvs[j] * rows_v[j, pl.ds(off, 16)]
                    acc_v[pl.ds(off, 16)] = acc
                    return 0

                lax.fori_loop(0, nchunks, chunk, 0, unroll=2)
                pltpu.sync_copy(acc_v, out_hbm.at[base + t])

            @pl.when(even)
            def _():
                compute(rows_a, sem_a, rows_b, sem_b)

            @pl.when(jnp.logical_not(even))
            def _():
                compute(rows_b, sem_b, rows_a, sem_a)

            return 0

        lax.fori_loop(0, TPW, body, 0)

    return k(w_up_table, idx2d, x2)


def _shared_swiglu_kernel(x_ref, w1_ref, w3_ref, w2_ref, o_ref):
    xb = x_ref[...]
    h1 = jnp.dot(xb, w1_ref[...].T, preferred_element_type=jnp.float32)
    h3 = jnp.dot(xb, w3_ref[...].T, preferred_element_type=jnp.float32)
    h = (h1 * jax.nn.sigmoid(h1)) * h3
    o_ref[...] = jnp.dot(h, w2_ref[...].T, preferred_element_type=jnp.float32)


def _shared_swiglu(x2d, s_w1, s_w2, s_w3, blk=256):
    bs, d = x2d.shape
    hid = s_w1.shape[0]
    grid = (bs // blk,)
    return pl.pallas_call(
        _shared_swiglu_kernel,
        grid=grid,
        in_specs=[
            pl.BlockSpec((blk, d), lambda i: (i, 0)),
            pl.BlockSpec((hid, d), lambda i: (0, 0)),
            pl.BlockSpec((hid, d), lambda i: (0, 0)),
            pl.BlockSpec((d, hid), lambda i: (0, 0)),
        ],
        out_specs=pl.BlockSpec((blk, d), lambda i: (i, 0)),
        out_shape=jax.ShapeDtypeStruct((bs, d), jnp.float32),
    )(x2d, s_w1, s_w3, s_w2)


def kernel(x, Wq, bq, bn_w, bn_b, bn_mean, bn_var, keys, w_down_table, w_up_table,
           a_w1, a_w2, a_w3, s_w1, s_w2, s_w3):
    b, t, d = x.shape
    bs = b * t
    x2d = x.reshape(bs, d)

    scale = bn_w / jnp.sqrt(bn_var + BN_EPS)
    wq_eff = (Wq * scale[:, None]).reshape(HEADS, K_DIM, d)
    b_eff = (bq * scale + bn_b - bn_mean * scale).reshape(HEADS, 1, K_DIM)

    s_full, cm8 = _score_chunks(x2d, wq_eff, b_eff, keys)
    chunks16 = jnp.pad(cm8, ((0, 0), (0, 0), (0, 16 - KNN))).reshape(NTH, 16)
    sc_scores, sc_idx = _sc_select(s_full.reshape(-1), chunks16)
    scores_h = sc_scores.reshape(HEADS, bs, 16)[:, :, :KNN]
    idx_h = sc_idx.reshape(HEADS, bs, 16)[:, :, :KNN]
    scores = scores_h.transpose(1, 0, 2).reshape(bs, KH)
    idx2d = idx_h.transpose(1, 0, 2).reshape(bs, KH)

    xm_all = _xm_all(x2d, w_down_table)
    tok = jnp.arange(bs, dtype=jnp.int32)[:, None]
    flat_idx = (((tok >> 3) * 200 + (idx2d >> 7)) * 1024
                + ((tok & 7) << 7) + (idx2d & 127)).reshape(-1)
    xm = _sc_gather_xm(xm_all.reshape(-1), flat_idx).reshape(bs, HEADS, KNN)

    xa = (jax.nn.silu(xm @ a_w1.T) * (xm @ a_w3.T)) @ a_w2.T
    x2 = (xa.reshape(bs, KH) * scores).astype(jnp.float32)

    out = _sc_combine(w_up_table, idx2d, x2)
    shared = _shared_swiglu(x2d, s_w1, s_w2, s_w3)
    return (out + shared).reshape(b, t, d)
